# Initial kernel scaffold; baseline (speedup 1.0000x reference)
import jax
import jax.numpy as jnp
from jax.experimental import pallas as pl


def kernel(hidden_states, gate_w, router_bias, w_gate, w_up, w_down):
    raise NotImplementedError("write your pallas kernel here")


# speedup vs baseline: 1.5082x; 1.5082x over previous
import functools

import jax
import jax.numpy as jnp
from jax import lax
from jax.experimental import pallas as pl
from jax.experimental.pallas import tpu as pltpu
from jax.experimental.pallas import tpu_sc as plsc

T = 2048
D = 1024
E = 8
K = 2
F = 512
ROUTED_SCALE = 1.0
EPS = 1e-20

LANES = 128
BM = 256
M_PAD = T * K + E * BM
NT = M_PAD // BM
CB = 256
NB = T // CB
NW = 32
TPW = T // NW


def _routing_body(x_ref, g_ref, b_ref, pos1_ref, pos2_ref, w1_ref, w2_ref,
                  texp_ref, tact_ref, c1_scr, c2_scr):
    x = x_ref[...]
    logits = lax.dot_general(x, g_ref[...], (((1,), (1,)), ((), ())),
                             preferred_element_type=jnp.float32)
    scores = jax.nn.sigmoid(logits)
    lane = lax.broadcasted_iota(jnp.int32, (T, LANES), 1)
    choice = scores + b_ref[...]
    m1 = jnp.max(choice, axis=1, keepdims=True)
    i1 = jnp.min(jnp.where(choice >= m1, lane, LANES), axis=1, keepdims=True)
    sel1 = lane == i1
    w1 = jnp.sum(jnp.where(sel1, scores, 0.0), axis=1, keepdims=True)
    choice2 = jnp.where(sel1, -1e30, choice)
    m2 = jnp.max(choice2, axis=1, keepdims=True)
    i2 = jnp.min(jnp.where(choice2 >= m2, lane, LANES), axis=1, keepdims=True)
    sel2 = lane == i2
    w2 = jnp.sum(jnp.where(sel2, scores, 0.0), axis=1, keepdims=True)
    den = w1 + w2 + EPS
    w1 = w1 / den * ROUTED_SCALE
    w2 = w2 / den * ROUTED_SCALE
    w1_ref[...] = jnp.broadcast_to(w1, (T, LANES))
    w2_ref[...] = jnp.broadcast_to(w2, (T, LANES))

    oh1 = sel1.astype(jnp.float32)
    oh2 = sel2.astype(jnp.float32)
    c1_scr[...] = oh1
    c2_scr[...] = oh2
    ltexc = (lax.broadcasted_iota(jnp.int32, (CB, CB), 0)
             > lax.broadcasted_iota(jnp.int32, (CB, CB), 1)).astype(jnp.float32)

    def blk(b, carry):
        car1, car2 = carry
        s = pl.ds(b * CB, CB)
        blk1 = c1_scr[s, :]
        blk2 = c2_scr[s, :]
        c1_scr[s, :] = lax.dot_general(
            ltexc, blk1, (((1,), (0,)), ((), ())),
            preferred_element_type=jnp.float32) + car1
        c2_scr[s, :] = lax.dot_general(
            ltexc, blk2, (((1,), (0,)), ((), ())),
            preferred_element_type=jnp.float32) + car2
        return (car1 + jnp.sum(blk1, axis=0, keepdims=True),
                car2 + jnp.sum(blk2, axis=0, keepdims=True))

    zero = jnp.zeros((1, LANES), jnp.float32)
    tot1, tot2 = lax.fori_loop(0, NB, blk, (zero, zero))

    counts = (tot1 + tot2).astype(jnp.int32)
    padded = lax.shift_left(lax.shift_right_logical(counts + (BM - 1), 8), 8)
    lte = (lax.broadcasted_iota(jnp.int32, (LANES, LANES), 0)
           < lax.broadcasted_iota(jnp.int32, (LANES, LANES), 1)).astype(jnp.float32)
    seg = lax.dot_general(padded.astype(jnp.float32), lte,
                          (((1,), (0,)), ((), ())),
                          preferred_element_type=jnp.float32)
    c1v = c1_scr[...]
    c2v = c2_scr[...]
    pos1 = jnp.sum((seg + c1v) * oh1, axis=1, keepdims=True)
    pos2 = jnp.sum((seg + tot1 + c2v) * oh2, axis=1, keepdims=True)
    pos1_ref[...] = jnp.broadcast_to(pos1.astype(jnp.int32), (T, LANES))
    pos2_ref[...] = jnp.broadcast_to(pos2.astype(jnp.int32), (T, LANES))

    segi = seg.astype(jnp.int32)
    tstart = lax.broadcasted_iota(jnp.int32, (32, LANES), 0) * BM
    act2 = (segi <= tstart) & (tstart < segi + counts)
    lane2 = lax.broadcasted_iota(jnp.int32, (32, LANES), 1)
    texp = jnp.sum(jnp.where(act2, lane2, 0), axis=1, keepdims=True)
    tact = jnp.sum(act2.astype(jnp.int32), axis=1, keepdims=True)
    texp_ref[...] = jnp.broadcast_to(texp, (32, LANES))
    tact_ref[...] = jnp.broadcast_to(tact, (32, LANES))


def _routing_call(x, gate_pad, bias_pad):
    return pl.pallas_call(
        _routing_body,
        out_shape=(
            jax.ShapeDtypeStruct((T, LANES), jnp.int32),
            jax.ShapeDtypeStruct((T, LANES), jnp.int32),
            jax.ShapeDtypeStruct((T, LANES), jnp.float32),
            jax.ShapeDtypeStruct((T, LANES), jnp.float32),
            jax.ShapeDtypeStruct((32, LANES), jnp.int32),
            jax.ShapeDtypeStruct((32, LANES), jnp.int32),
        ),
        scratch_shapes=[
            pltpu.VMEM((T, LANES), jnp.float32),
            pltpu.VMEM((T, LANES), jnp.float32),
        ],
    )(x, gate_pad, bias_pad)


def _gmm_body(texp_ref, tact_ref, xs_ref, wgu_ref, wd_ref, y_ref):
    i = pl.program_id(0)

    @pl.when(tact_ref[i] == 1)
    def _():
        xt = xs_ref[...]
        z = jnp.dot(xt, wgu_ref[0], preferred_element_type=jnp.float32)
        g = z[:, :F]
        u = z[:, F:]
        act = g * jax.nn.sigmoid(g) * u
        y_ref[...] = jnp.dot(act, wd_ref[0], preferred_element_type=jnp.float32)


def _gmm_call(texp, tact, xs, wgu, wd):
    grid_spec = pltpu.PrefetchScalarGridSpec(
        num_scalar_prefetch=2,
        grid=(NT,),
        in_specs=[
            pl.BlockSpec((BM, D), lambda i, texp, tact: (i, 0)),
            pl.BlockSpec((1, D, 2 * F), lambda i, texp, tact: (texp[i], 0, 0)),
            pl.BlockSpec((1, F, D), lambda i, texp, tact: (texp[i], 0, 0)),
        ],
        out_specs=pl.BlockSpec((BM, D), lambda i, texp, tact: (i, 0)),
    )
    return pl.pallas_call(
        _gmm_body,
        grid_spec=grid_spec,
        out_shape=jax.ShapeDtypeStruct((M_PAD, D), jnp.float32),
    )(texp, tact, xs, wgu, wd)


def _dispatch_body(x_hbm, pos_hbm, xs_hbm, xbuf, idx0, idx1, sem):
    w = lax.axis_index("s") * 2 + lax.axis_index("c")
    base = w * TPW
    pltpu.sync_copy(x_hbm.at[pl.ds(base, TPW)], xbuf)
    pltpu.sync_copy(pos_hbm.at[w, 0], idx0)
    pltpu.sync_copy(pos_hbm.at[w, 1], idx1)
    pltpu.async_copy(xbuf, xs_hbm.at[idx0], sem).wait()
    pltpu.async_copy(xbuf, xs_hbm.at[idx1], sem).wait()


@functools.cache
def _dispatch_kernel():
    mesh = plsc.VectorSubcoreMesh(core_axis_name="c", subcore_axis_name="s")
    return pl.kernel(
        _dispatch_body,
        out_type=jax.ShapeDtypeStruct((M_PAD, D), jnp.float32),
        mesh=mesh,
        scratch_types=[
            pltpu.VMEM((TPW, D), jnp.float32),
            pltpu.VMEM((TPW,), jnp.int32),
            pltpu.VMEM((TPW,), jnp.int32),
            pltpu.SemaphoreType.DMA,
        ],
    )


def _combine_body(y_hbm, pos_hbm, y0_hbm, y1_hbm, ybuf, idx, sem):
    w = lax.axis_index("s") * 2 + lax.axis_index("c")
    base = w * TPW
    pltpu.sync_copy(pos_hbm.at[w, 0], idx)
    pltpu.async_copy(y_hbm.at[idx], ybuf, sem).wait()
    pltpu.sync_copy(ybuf, y0_hbm.at[pl.ds(base, TPW)])
    pltpu.sync_copy(pos_hbm.at[w, 1], idx)
    pltpu.async_copy(y_hbm.at[idx], ybuf, sem).wait()
    pltpu.sync_copy(ybuf, y1_hbm.at[pl.ds(base, TPW)])


@functools.cache
def _combine_kernel():
    mesh = plsc.VectorSubcoreMesh(core_axis_name="c", subcore_axis_name="s")
    return pl.kernel(
        _combine_body,
        out_type=(
            jax.ShapeDtypeStruct((T, D), jnp.float32),
            jax.ShapeDtypeStruct((T, D), jnp.float32),
        ),
        mesh=mesh,
        scratch_types=[
            pltpu.VMEM((TPW, D), jnp.float32),
            pltpu.VMEM((TPW,), jnp.int32),
            pltpu.SemaphoreType.DMA,
        ],
    )


def _final_body(y0_ref, y1_ref, w1_ref, w2_ref, o_ref):
    o_ref[...] = w1_ref[:, :1] * y0_ref[...] + w2_ref[:, :1] * y1_ref[...]


def _final_call(y0, y1, w1b, w2b):
    return pl.pallas_call(
        _final_body,
        grid=(T // BM,),
        in_specs=[
            pl.BlockSpec((BM, D), lambda i: (i, 0)),
            pl.BlockSpec((BM, D), lambda i: (i, 0)),
            pl.BlockSpec((BM, LANES), lambda i: (i, 0)),
            pl.BlockSpec((BM, LANES), lambda i: (i, 0)),
        ],
        out_specs=pl.BlockSpec((BM, D), lambda i: (i, 0)),
        out_shape=jax.ShapeDtypeStruct((T, D), jnp.float32),
    )(y0, y1, w1b, w2b)


def kernel(hidden_states, gate_w, router_bias, w_gate, w_up, w_down):
    x = hidden_states.astype(jnp.float32)
    gate_pad = jnp.zeros((LANES, D), jnp.float32).at[:E].set(gate_w)
    bias_pad = jnp.full((1, LANES), -1e30, jnp.float32).at[0, :E].set(router_bias)
    wgu = jnp.concatenate([w_gate, w_up], axis=2)

    pos1b, pos2b, w1b, w2b, texp_b, tact_b = _routing_call(x, gate_pad, bias_pad)
    texp = texp_b[:NT, 0]
    tact = tact_b[:NT, 0]
    pos_sc = jnp.stack(
        [pos1b[:, 0].reshape(NW, TPW), pos2b[:, 0].reshape(NW, TPW)], axis=1)

    xs = _dispatch_kernel()(x, pos_sc)
    y = _gmm_call(texp, tact, xs, wgu, w_down)
    y0, y1 = _combine_kernel()(y, pos_sc)
    return _final_call(y0, y1, w1b, w2b)
